# baseline (speedup 1.0000x reference)
import functools

import jax
import jax.numpy as jnp
from jax import lax
from jax.experimental import pallas as pl
from jax.experimental.pallas import tpu as pltpu
from jax.experimental.pallas import tpu_sc as plsc

_NT = 7
_N = 50000
_D = 128
_E = 80000
_BLK = 2000

_NSUB = 16
_EPS = _E // _NSUB
_EPS_PAD = _EPS + 24
_NVREG2 = (_EPS + 31) // 32
_NPASS = 3
_RSEG = 8448
_CORE_ROWS = _NPASS * _RSEG
_NPAD = 2 * _CORE_ROWS
_GCH = 128
_LROWS = 42
_RPS = _RSEG // _NSUB


def _row(pos):
    return lax.shift_right_arithmetic(pos, 7)


def _col(pos):
    return lax.bitwise_and(pos, _GCH - 1)


def _seg_sums(h_flat, src, dst, zeros):
    mesh = plsc.VectorSubcoreMesh(core_axis_name="c", subcore_axis_name="s")

    @functools.partial(
        pl.kernel,
        mesh=mesh,
        compiler_params=pltpu.CompilerParams(needs_layout_passes=False),
        out_type=jax.ShapeDtypeStruct((_NPAD, _D), jnp.float32),
        scratch_types=[
            pltpu.VMEM((_EPS_PAD,), jnp.int32),
            pltpu.VMEM((_EPS_PAD,), jnp.int32),
            pltpu.VMEM((_LROWS, _GCH), jnp.int32),
            pltpu.VMEM((_LROWS, _GCH), jnp.int32),
            pltpu.VMEM((_GCH, _D), jnp.float32),
            pltpu.VMEM((_GCH, _D), jnp.float32),
            pltpu.VMEM_SHARED((_RSEG + 16, _D), jnp.float32),
            pltpu.SemaphoreType.DMA,
            pltpu.SemaphoreType.DMA,
            pltpu.SemaphoreType.DMA,
            pltpu.SemaphoreType.DMA,
        ],
    )
    def body(src_hbm, dst_hbm, h_hbm, z_hbm, a_hbm,
             src_st, dst_st, ssrc, sdst, rowbuf0, rowbuf1, acc,
             sem0, sem1, semw, semz):
        c = lax.axis_index("c")
        s = lax.axis_index("s")
        clo = c * _CORE_ROWS
        ebase = s * _EPS
        lanes = lax.iota(jnp.int32, 16)
        zb = s * _RPS

        pltpu.async_copy(z_hbm.at[pl.ds(zb, _RPS)],
                         acc.at[pl.ds(zb, _RPS)], semz).wait()
        plsc.subcore_barrier()

        def scan(plo, toff):
            def scan_body(i, cnt):
                off = i * 32
                da = dst_st[pl.ds(off, 16)]
                sa = src_st[pl.ds(off, 16)]
                db = dst_st[pl.ds(off + 16, 16)]
                sb = src_st[pl.ds(off + 16, 16)]
                va = (off + lanes) < _EPS
                vb = (off + 16 + lanes) < _EPS
                dla = da - plo
                dlb = db - plo
                ma = (dla >= 0) & (dla < _RSEG) & va
                mb = (dlb >= 0) & (dlb < _RSEG) & vb
                ia = plsc.cumsum(ma.astype(jnp.int32))
                ib = plsc.cumsum(mb.astype(jnp.int32))
                pa = cnt + ia - 1
                plsc.store_scatter(ssrc, [_row(pa), _col(pa)],
                                   sa + toff, mask=ma)
                plsc.store_scatter(sdst, [_row(pa), _col(pa)],
                                   dla, mask=ma)
                cmid = cnt + ia[15]
                pb = cmid + ib - 1
                plsc.store_scatter(ssrc, [_row(pb), _col(pb)],
                                   sb + toff, mask=mb)
                plsc.store_scatter(sdst, [_row(pb), _col(pb)],
                                   dlb, mask=mb)
                return cmid + ib[15]

            cnt = lax.fori_loop(0, _NVREG2, scan_body,
                                jnp.zeros((), jnp.int32))

            for k in range(_GCH // 16):
                pp = cnt + k * 16 + lanes
                plsc.store_scatter(ssrc, [_row(pp), _col(pp)], lanes)
                plsc.store_scatter(sdst, [_row(pp), _col(pp)],
                                   _RSEG + lanes)
            return cnt

        def chunks(cnt):
            nch = (cnt + _GCH - 1) // _GCH

            @pl.when(nch > 0)
            def _():
                pltpu.async_copy(h_hbm.at[ssrc.at[0]], rowbuf0, sem0)

            def pair_body(jj, _):
                for b, bufc, semc, bufn, semn in (
                        (0, rowbuf0, sem0, rowbuf1, sem1),
                        (1, rowbuf1, sem1, rowbuf0, sem0)):
                    j = jj * 2 + b

                    @pl.when(j < nch)
                    def _():
                        pltpu.make_async_copy(
                            h_hbm.at[ssrc.at[j]], bufc, semc).wait()

                        @pl.when(j + 1 < nch)
                        def _():
                            pltpu.async_copy(
                                h_hbm.at[ssrc.at[j + 1]], bufn, semn)

                        pltpu.sync_copy(bufc, acc.at[sdst.at[j]],
                                        add=True)
                return 0

            lax.fori_loop(0, (nch + 1) // 2, pair_body, 0)

        def pass_body(p, _):
            plo = clo + p * _RSEG

            def type_body(t, _):
                pltpu.sync_copy(src_hbm.at[pl.ds(t * _E + ebase, _EPS)],
                                src_st.at[pl.ds(0, _EPS)])
                pltpu.sync_copy(dst_hbm.at[pl.ds(t * _E + ebase, _EPS)],
                                dst_st.at[pl.ds(0, _EPS)])
                cnt = scan(plo, t * _N)
                chunks(cnt)
                return 0

            lax.fori_loop(0, _NT, type_body, 0)
            plsc.subcore_barrier()
            w = pltpu.async_copy(acc.at[pl.ds(zb, _RPS)],
                                 a_hbm.at[pl.ds(plo + zb, _RPS)], semw)
            w.wait()
            pltpu.async_copy(z_hbm.at[pl.ds(zb, _RPS)],
                             acc.at[pl.ds(zb, _RPS)], semz).wait()
            plsc.subcore_barrier()
            return 0

        lax.fori_loop(0, _NPASS, pass_body, 0)

    return body(src, dst, h_flat, zeros)


def _hmul_body(x_ref, w_ref, o_ref):
    o_ref[0] = jnp.dot(x_ref[...], w_ref[0],
                       preferred_element_type=jnp.float32)


def _hmul(x, Wnbr):
    return pl.pallas_call(
        _hmul_body,
        grid=(_NT, _N // _BLK),
        in_specs=[
            pl.BlockSpec((_BLK, _D), lambda t, i: (i, 0)),
            pl.BlockSpec((1, _D, _D), lambda t, i: (t, 0, 0)),
        ],
        out_specs=pl.BlockSpec((1, _BLK, _D), lambda t, i: (t, i, 0)),
        out_shape=jax.ShapeDtypeStruct((_NT, _N, _D), jnp.float32),
    )(x, Wnbr)


def _combine_body(x_ref, a_ref, ws_ref, b_ref, o_ref):
    xw = jnp.dot(x_ref[...], jnp.sum(ws_ref[...], axis=0),
                 preferred_element_type=jnp.float32)
    o_ref[...] = (xw + a_ref[...] + jnp.sum(b_ref[...], axis=0)) * (1.0 / _NT)


def _combine(x, A, Wself, B):
    return pl.pallas_call(
        _combine_body,
        grid=(_N // _BLK,),
        in_specs=[
            pl.BlockSpec((_BLK, _D), lambda i: (i, 0)),
            pl.BlockSpec((_BLK, _D), lambda i: (i, 0)),
            pl.BlockSpec((_NT, _D, _D), lambda i: (0, 0, 0)),
            pl.BlockSpec((_NT, _D), lambda i: (0, 0)),
        ],
        out_specs=pl.BlockSpec((_BLK, _D), lambda i: (i, 0)),
        out_shape=jax.ShapeDtypeStruct((_N, _D), jnp.float32),
    )(x, A, Wself, B)


def kernel(x, edge_index_candidate2candidate, W_self_candidate2candidate, W_nbr_candidate2candidate, b_candidate2candidate, edge_index_candidate2document, W_self_candidate2document, W_nbr_candidate2document, b_candidate2document, edge_index_candidate2entity, W_self_candidate2entity, W_nbr_candidate2entity, b_candidate2entity, edge_index_codocument, W_self_codocument, W_nbr_codocument, b_codocument, edge_index_comention, W_self_comention, W_nbr_comention, b_comention, edge_index_document2entity, W_self_document2entity, W_nbr_document2entity, b_document2entity, edge_index_entity, W_self_entity, W_nbr_entity, b_entity):
    edges = [edge_index_candidate2candidate, edge_index_candidate2document,
             edge_index_candidate2entity, edge_index_codocument,
             edge_index_comention, edge_index_document2entity,
             edge_index_entity]
    Wself = jnp.stack([W_self_candidate2candidate, W_self_candidate2document,
                       W_self_candidate2entity, W_self_codocument,
                       W_self_comention, W_self_document2entity,
                       W_self_entity])
    Wnbr = jnp.stack([W_nbr_candidate2candidate, W_nbr_candidate2document,
                      W_nbr_candidate2entity, W_nbr_codocument,
                      W_nbr_comention, W_nbr_document2entity,
                      W_nbr_entity])
    B = jnp.stack([b_candidate2candidate, b_candidate2document,
                   b_candidate2entity, b_codocument, b_comention,
                   b_document2entity, b_entity])
    SRC = jnp.concatenate([e[0] for e in edges])
    DST = jnp.concatenate([e[1] for e in edges])
    zeros = jnp.zeros((_RSEG, _D), jnp.float32)

    H = _hmul(x, Wnbr).reshape(_NT * _N, _D)
    A = _seg_sums(H, SRC, DST, zeros)
    return _combine(x, A, Wself, B)


# speedup vs baseline: 1.0184x; 1.0184x over previous
import functools

import jax
import jax.numpy as jnp
from jax import lax
from jax.experimental import pallas as pl
from jax.experimental.pallas import tpu as pltpu
from jax.experimental.pallas import tpu_sc as plsc

_NT = 7
_N = 50000
_D = 128
_E = 80000
_BLK = 2000

_NSUB = 16
_EPS = _E // _NSUB
_EPS_PAD = _EPS + 56
_NVREG4 = (_EPS + 63) // 64
_NPASS = 3
_RSEG = 8448
_CORE_ROWS = _NPASS * _RSEG
_NPAD = 2 * _CORE_ROWS
_GCH = 128
_LROWS = 42
_RPS = _RSEG // _NSUB


def _row(pos):
    return lax.shift_right_arithmetic(pos, 7)


def _col(pos):
    return lax.bitwise_and(pos, _GCH - 1)


def _seg_sums(x, src, dst, zeros):
    mesh = plsc.VectorSubcoreMesh(core_axis_name="c", subcore_axis_name="s")

    @functools.partial(
        pl.kernel,
        mesh=mesh,
        compiler_params=pltpu.CompilerParams(needs_layout_passes=False),
        out_type=jax.ShapeDtypeStruct((_NT, _NPAD, _D), jnp.float32),
        scratch_types=[
            pltpu.VMEM((_EPS_PAD,), jnp.int32),
            pltpu.VMEM((_EPS_PAD,), jnp.int32),
            pltpu.VMEM((_LROWS, _GCH), jnp.int32),
            pltpu.VMEM((_LROWS, _GCH), jnp.int32),
            pltpu.VMEM((_GCH, _D), jnp.float32),
            pltpu.VMEM((_GCH, _D), jnp.float32),
            pltpu.VMEM_SHARED((_RSEG + 16, _D), jnp.float32),
            pltpu.SemaphoreType.DMA,
            pltpu.SemaphoreType.DMA,
            pltpu.SemaphoreType.DMA,
            pltpu.SemaphoreType.DMA,
            pltpu.SemaphoreType.DMA,
            pltpu.SemaphoreType.DMA,
        ],
    )
    def body(src_hbm, dst_hbm, x_hbm, z_hbm, g_hbm,
             src_st, dst_st, ssrc, sdst, rowbuf0, rowbuf1, acc,
             sem0, sem1, sems0, sems1, semw, semz):
        c = lax.axis_index("c")
        s = lax.axis_index("s")
        clo = c * _CORE_ROWS
        ebase = s * _EPS
        lanes = lax.iota(jnp.int32, 16)
        zb = s * _RPS

        pltpu.async_copy(z_hbm.at[pl.ds(zb, _RPS)],
                         acc.at[pl.ds(zb, _RPS)], semz).wait()
        plsc.subcore_barrier()

        def scan(plo):
            def scan_body(i, cnt):
                off = i * 64
                for k in range(4):
                    d = dst_st[pl.ds(off + k * 16, 16)]
                    sv = src_st[pl.ds(off + k * 16, 16)]
                    v = (off + k * 16 + lanes) < _EPS
                    dl = d - plo
                    m = (dl >= 0) & (dl < _RSEG) & v
                    inc = plsc.cumsum(m.astype(jnp.int32))
                    pos = cnt + inc - 1
                    plsc.store_scatter(ssrc, [_row(pos), _col(pos)],
                                       sv, mask=m)
                    plsc.store_scatter(sdst, [_row(pos), _col(pos)],
                                       dl, mask=m)
                    cnt = cnt + inc[15]
                return cnt

            cnt = lax.fori_loop(0, _NVREG4, scan_body,
                                jnp.zeros((), jnp.int32))

            for k in range(_GCH // 16):
                pp = cnt + k * 16 + lanes
                plsc.store_scatter(ssrc, [_row(pp), _col(pp)], lanes)
                plsc.store_scatter(sdst, [_row(pp), _col(pp)],
                                   _RSEG + lanes)
            return cnt

        def chunks(cnt):
            nch = (cnt + _GCH - 1) // _GCH

            @pl.when(nch > 0)
            def _():
                pltpu.async_copy(x_hbm.at[ssrc.at[0]], rowbuf0, sem0)

            def pair_body(jj, _):
                for b, bufc, semc, bufn, semn in (
                        (0, rowbuf0, sem0, rowbuf1, sem1),
                        (1, rowbuf1, sem1, rowbuf0, sem0)):
                    j = jj * 2 + b

                    @pl.when(j < nch)
                    def _():
                        pltpu.make_async_copy(
                            x_hbm.at[ssrc.at[j]], bufc, semc).wait()

                        @pl.when(j + 1 < nch)
                        def _():
                            pltpu.async_copy(
                                x_hbm.at[ssrc.at[j + 1]], bufn, semn)

                        pltpu.sync_copy(bufc, acc.at[sdst.at[j]],
                                        add=True)
                return 0

            lax.fori_loop(0, (nch + 1) // 2, pair_body, 0)

        def type_body(t, _):
            pltpu.sync_copy(src_hbm.at[pl.ds(t * _E + ebase, _EPS)],
                            src_st.at[pl.ds(0, _EPS)])
            pltpu.sync_copy(dst_hbm.at[pl.ds(t * _E + ebase, _EPS)],
                            dst_st.at[pl.ds(0, _EPS)])

            cnt = scan(clo)
            for p in range(_NPASS):
                chunks(cnt)
                plsc.subcore_barrier()
                w = pltpu.async_copy(
                    acc.at[pl.ds(zb, _RPS)],
                    g_hbm.at[t, pl.ds(clo + p * _RSEG + zb, _RPS)], semw)
                if p + 1 < _NPASS:
                    cnt = scan(clo + (p + 1) * _RSEG)
                w.wait()
                pltpu.async_copy(z_hbm.at[pl.ds(zb, _RPS)],
                                 acc.at[pl.ds(zb, _RPS)], semz).wait()
                plsc.subcore_barrier()
            return 0

        lax.fori_loop(0, _NT, type_body, 0)

    return body(src, dst, x, zeros)


def _combine_body(x_ref, g_ref, ws_ref, wn_ref, b_ref, o_ref):
    x = x_ref[...]
    acc = jnp.dot(x, jnp.sum(ws_ref[...], axis=0),
                  preferred_element_type=jnp.float32)
    g = g_ref[...]
    for t in range(_NT):
        acc = acc + jnp.dot(g[t], wn_ref[t],
                            preferred_element_type=jnp.float32)
    o_ref[...] = (acc + jnp.sum(b_ref[...], axis=0)) * (1.0 / _NT)


def _combine(x, G, Wself, Wnbr, B):
    grid = (_N // _BLK,)
    return pl.pallas_call(
        _combine_body,
        grid=grid,
        in_specs=[
            pl.BlockSpec((_BLK, _D), lambda i: (i, 0)),
            pl.BlockSpec((_NT, _BLK, _D), lambda i: (0, i, 0)),
            pl.BlockSpec((_NT, _D, _D), lambda i: (0, 0, 0)),
            pl.BlockSpec((_NT, _D, _D), lambda i: (0, 0, 0)),
            pl.BlockSpec((_NT, _D), lambda i: (0, 0)),
        ],
        out_specs=pl.BlockSpec((_BLK, _D), lambda i: (i, 0)),
        out_shape=jax.ShapeDtypeStruct((_N, _D), jnp.float32),
    )(x, G, Wself, Wnbr, B)


def kernel(x, edge_index_candidate2candidate, W_self_candidate2candidate, W_nbr_candidate2candidate, b_candidate2candidate, edge_index_candidate2document, W_self_candidate2document, W_nbr_candidate2document, b_candidate2document, edge_index_candidate2entity, W_self_candidate2entity, W_nbr_candidate2entity, b_candidate2entity, edge_index_codocument, W_self_codocument, W_nbr_codocument, b_codocument, edge_index_comention, W_self_comention, W_nbr_comention, b_comention, edge_index_document2entity, W_self_document2entity, W_nbr_document2entity, b_document2entity, edge_index_entity, W_self_entity, W_nbr_entity, b_entity):
    edges = [edge_index_candidate2candidate, edge_index_candidate2document,
             edge_index_candidate2entity, edge_index_codocument,
             edge_index_comention, edge_index_document2entity,
             edge_index_entity]
    Wself = jnp.stack([W_self_candidate2candidate, W_self_candidate2document,
                       W_self_candidate2entity, W_self_codocument,
                       W_self_comention, W_self_document2entity,
                       W_self_entity])
    Wnbr = jnp.stack([W_nbr_candidate2candidate, W_nbr_candidate2document,
                      W_nbr_candidate2entity, W_nbr_codocument,
                      W_nbr_comention, W_nbr_document2entity,
                      W_nbr_entity])
    B = jnp.stack([b_candidate2candidate, b_candidate2document,
                   b_candidate2entity, b_codocument, b_comention,
                   b_document2entity, b_entity])
    SRC = jnp.concatenate([e[0] for e in edges])
    DST = jnp.concatenate([e[1] for e in edges])
    zeros = jnp.zeros((_RSEG, _D), jnp.float32)

    G = _seg_sums(x, SRC, DST, zeros)
    return _combine(x, G, Wself, Wnbr, B)


# speedup vs baseline: 1.0281x; 1.0095x over previous
import functools

import jax
import jax.numpy as jnp
from jax import lax
from jax.experimental import pallas as pl
from jax.experimental.pallas import tpu as pltpu
from jax.experimental.pallas import tpu_sc as plsc

_NT = 7
_N = 50000
_D = 128
_E = 80000
_BLK = 2000

_NSUB = 16
_EPS = _E // _NSUB
_EPS_PAD = _EPS + 24
_NVREG2 = (_EPS + 31) // 32
_NPASS = 3
_RSEG = 8448
_CORE_ROWS = _NPASS * _RSEG
_NPAD = 2 * _CORE_ROWS
_GCH = 128
_LROWS = 42
_RPS = _RSEG // _NSUB


def _row(pos):
    return lax.shift_right_arithmetic(pos, 7)


def _col(pos):
    return lax.bitwise_and(pos, _GCH - 1)


def _seg_sums(x, src, dst, zeros):
    mesh = plsc.VectorSubcoreMesh(core_axis_name="c", subcore_axis_name="s")

    @functools.partial(
        pl.kernel,
        mesh=mesh,
        compiler_params=pltpu.CompilerParams(needs_layout_passes=False),
        out_type=jax.ShapeDtypeStruct((_NT, _NPAD, _D), jnp.float32),
        scratch_types=[
            pltpu.VMEM((_EPS_PAD,), jnp.int32),
            pltpu.VMEM((_EPS_PAD,), jnp.int32),
            pltpu.VMEM((_LROWS, _GCH), jnp.int32),
            pltpu.VMEM((_LROWS, _GCH), jnp.int32),
            pltpu.VMEM((_GCH, _D), jnp.float32),
            pltpu.VMEM((_GCH, _D), jnp.float32),
            pltpu.VMEM_SHARED((_RSEG + 16, _D), jnp.float32),
            pltpu.SemaphoreType.DMA,
            pltpu.SemaphoreType.DMA,
            pltpu.SemaphoreType.DMA,
            pltpu.SemaphoreType.DMA,
            pltpu.SemaphoreType.DMA,
            pltpu.SemaphoreType.DMA,
        ],
    )
    def body(src_hbm, dst_hbm, x_hbm, z_hbm, g_hbm,
             src_st, dst_st, ssrc, sdst, rowbuf0, rowbuf1, acc,
             sem0, sem1, sems0, sems1, semw, semz):
        c = lax.axis_index("c")
        s = lax.axis_index("s")
        clo = c * _CORE_ROWS
        ebase = s * _EPS
        lanes = lax.iota(jnp.int32, 16)
        zb = s * _RPS

        pltpu.async_copy(z_hbm.at[pl.ds(zb, _RPS)],
                         acc.at[pl.ds(zb, _RPS)], semz).wait()
        plsc.subcore_barrier()

        def scan(plo):
            def scan_body(i, cnt):
                off = i * 32
                da = dst_st[pl.ds(off, 16)]
                sa = src_st[pl.ds(off, 16)]
                db = dst_st[pl.ds(off + 16, 16)]
                sb = src_st[pl.ds(off + 16, 16)]
                va = (off + lanes) < _EPS
                vb = (off + 16 + lanes) < _EPS
                dla = da - plo
                dlb = db - plo
                ma = (dla >= 0) & (dla < _RSEG) & va
                mb = (dlb >= 0) & (dlb < _RSEG) & vb
                ia = plsc.cumsum(ma.astype(jnp.int32))
                ib = plsc.cumsum(mb.astype(jnp.int32))
                pa = cnt + ia - 1
                plsc.store_scatter(ssrc, [_row(pa), _col(pa)],
                                   sa, mask=ma)
                plsc.store_scatter(sdst, [_row(pa), _col(pa)],
                                   dla, mask=ma)
                cmid = cnt + ia[15]
                pb = cmid + ib - 1
                plsc.store_scatter(ssrc, [_row(pb), _col(pb)],
                                   sb, mask=mb)
                plsc.store_scatter(sdst, [_row(pb), _col(pb)],
                                   dlb, mask=mb)
                return cmid + ib[15]

            cnt = lax.fori_loop(0, _NVREG2, scan_body,
                                jnp.zeros((), jnp.int32))

            for k in range(_GCH // 16):
                pp = cnt + k * 16 + lanes
                plsc.store_scatter(ssrc, [_row(pp), _col(pp)], lanes)
                plsc.store_scatter(sdst, [_row(pp), _col(pp)],
                                   _RSEG + lanes)
            return cnt

        def chunks(cnt):
            nch = (cnt + _GCH - 1) // _GCH

            @pl.when(nch > 0)
            def _():
                pltpu.async_copy(x_hbm.at[ssrc.at[0]], rowbuf0, sem0)

            def pair_body(jj, _):
                for b, bufc, semc, bufn, semn in (
                        (0, rowbuf0, sem0, rowbuf1, sem1),
                        (1, rowbuf1, sem1, rowbuf0, sem0)):
                    j = jj * 2 + b

                    @pl.when(j < nch)
                    def _():
                        pltpu.make_async_copy(
                            x_hbm.at[ssrc.at[j]], bufc, semc).wait()

                        @pl.when(j + 1 < nch)
                        def _():
                            pltpu.async_copy(
                                x_hbm.at[ssrc.at[j + 1]], bufn, semn)

                        pltpu.sync_copy(bufc, acc.at[sdst.at[j]],
                                        add=True)
                return 0

            lax.fori_loop(0, (nch + 1) // 2, pair_body, 0)

        def type_body(t, _):
            pltpu.sync_copy(src_hbm.at[pl.ds(t * _E + ebase, _EPS)],
                            src_st.at[pl.ds(0, _EPS)])
            pltpu.sync_copy(dst_hbm.at[pl.ds(t * _E + ebase, _EPS)],
                            dst_st.at[pl.ds(0, _EPS)])

            cnt = scan(clo)
            for p in range(_NPASS):
                chunks(cnt)
                plsc.subcore_barrier()
                w = pltpu.async_copy(
                    acc.at[pl.ds(zb, _RPS)],
                    g_hbm.at[t, pl.ds(clo + p * _RSEG + zb, _RPS)], semw)
                if p + 1 < _NPASS:
                    cnt = scan(clo + (p + 1) * _RSEG)
                w.wait()
                pltpu.async_copy(z_hbm.at[pl.ds(zb, _RPS)],
                                 acc.at[pl.ds(zb, _RPS)], semz).wait()
                plsc.subcore_barrier()
            return 0

        lax.fori_loop(0, _NT, type_body, 0)

    return body(src, dst, x, zeros)


def _combine_body(x_ref, g_ref, ws_ref, wn_ref, b_ref, o_ref):
    x = x_ref[...]
    acc = jnp.dot(x, jnp.sum(ws_ref[...], axis=0),
                  preferred_element_type=jnp.float32)
    g = g_ref[...]
    for t in range(_NT):
        acc = acc + jnp.dot(g[t], wn_ref[t],
                            preferred_element_type=jnp.float32)
    o_ref[...] = (acc + jnp.sum(b_ref[...], axis=0)) * (1.0 / _NT)


def _combine(x, G, Wself, Wnbr, B):
    grid = (_N // _BLK,)
    return pl.pallas_call(
        _combine_body,
        grid=grid,
        in_specs=[
            pl.BlockSpec((_BLK, _D), lambda i: (i, 0)),
            pl.BlockSpec((_NT, _BLK, _D), lambda i: (0, i, 0)),
            pl.BlockSpec((_NT, _D, _D), lambda i: (0, 0, 0)),
            pl.BlockSpec((_NT, _D, _D), lambda i: (0, 0, 0)),
            pl.BlockSpec((_NT, _D), lambda i: (0, 0)),
        ],
        out_specs=pl.BlockSpec((_BLK, _D), lambda i: (i, 0)),
        out_shape=jax.ShapeDtypeStruct((_N, _D), jnp.float32),
    )(x, G, Wself, Wnbr, B)


def kernel(x, edge_index_candidate2candidate, W_self_candidate2candidate, W_nbr_candidate2candidate, b_candidate2candidate, edge_index_candidate2document, W_self_candidate2document, W_nbr_candidate2document, b_candidate2document, edge_index_candidate2entity, W_self_candidate2entity, W_nbr_candidate2entity, b_candidate2entity, edge_index_codocument, W_self_codocument, W_nbr_codocument, b_codocument, edge_index_comention, W_self_comention, W_nbr_comention, b_comention, edge_index_document2entity, W_self_document2entity, W_nbr_document2entity, b_document2entity, edge_index_entity, W_self_entity, W_nbr_entity, b_entity):
    edges = [edge_index_candidate2candidate, edge_index_candidate2document,
             edge_index_candidate2entity, edge_index_codocument,
             edge_index_comention, edge_index_document2entity,
             edge_index_entity]
    Wself = jnp.stack([W_self_candidate2candidate, W_self_candidate2document,
                       W_self_candidate2entity, W_self_codocument,
                       W_self_comention, W_self_document2entity,
                       W_self_entity])
    Wnbr = jnp.stack([W_nbr_candidate2candidate, W_nbr_candidate2document,
                      W_nbr_candidate2entity, W_nbr_codocument,
                      W_nbr_comention, W_nbr_document2entity,
                      W_nbr_entity])
    B = jnp.stack([b_candidate2candidate, b_candidate2document,
                   b_candidate2entity, b_codocument, b_comention,
                   b_document2entity, b_entity])
    SRC = jnp.concatenate([e[0] for e in edges])
    DST = jnp.concatenate([e[1] for e in edges])
    zeros = jnp.zeros((_RSEG, _D), jnp.float32)

    G = _seg_sums(x, SRC, DST, zeros)
    return _combine(x, G, Wself, Wnbr, B)
